# baseline (speedup 1.0000x reference)
import functools

import jax
import jax.numpy as jnp
from jax import lax
from jax.experimental import pallas as pl
from jax.experimental.pallas import tpu as pltpu

_VMEM_LIMIT_BYTES = 60 << 20


def _cbn_kernel(y_ref, x_ref, e0_ref, e1_ref, o_ref, g_sc, b_sc,
                *, eps, inv_n, ct, nb):
    B = x_ref.shape[0]
    ci = pl.program_id(0)

    @pl.when(ci == 0)
    def _gather():
        def body(b, carry):
            yb = y_ref[b]
            r0 = e0_ref[pl.ds(yb, 1), :]
            r1 = e1_ref[pl.ds(yb, 1), :]
            for k in range(nb):
                g_sc[k, pl.ds(b, 1), :] = 1.0 + r0[:, k * ct:(k + 1) * ct]
                b_sc[k, pl.ds(b, 1), :] = r1[:, k * ct:(k + 1) * ct]
            return carry
        lax.fori_loop(0, B, body, 0)

    x = x_ref[...]

    s1 = jnp.sum(x, axis=2, keepdims=True)
    s2 = jnp.sum(x * x, axis=2, keepdims=True)
    mean = jnp.sum(s1, axis=0, keepdims=True) * inv_n
    ex2 = jnp.sum(s2, axis=0, keepdims=True) * inv_n
    var = jnp.maximum(ex2 - mean * mean, 0.0)
    inv_std = lax.rsqrt(var + eps)

    g3 = g_sc[ci].reshape(B, ct, 1)
    b3 = b_sc[ci].reshape(B, ct, 1)
    scale = g3 * inv_std
    shift = b3 - mean * scale

    o_ref[...] = x * scale + shift


def _pick_channel_tile(B, C, HW, itemsize, target_bytes=4 << 20):
    if C % 8 != 0:
        return C
    per_channel = max(B * HW * itemsize, 1)
    max_ct = (target_bytes // per_channel) // 8 * 8
    max_ct = max(8, min(int(max_ct), C))
    for cand in range(max_ct, 7, -8):
        if C % cand == 0:
            return cand
    return 8


@functools.partial(jax.jit, static_argnames=("eps",))
def _cbn(x, y, embed0, embed1, *, eps=1e-4):
    B, C, H, W = x.shape
    HW = H * W
    x3 = x.reshape(B, C, HW)
    ct = _pick_channel_tile(B, C, HW, x.dtype.itemsize)
    nb = C // ct
    e0 = embed0.astype(jnp.float32)
    e1 = embed1.astype(jnp.float32)
    NC = e0.shape[0]
    kern = functools.partial(_cbn_kernel, eps=float(eps),
                             inv_n=1.0 / float(B * HW), ct=ct, nb=nb)
    out3 = pl.pallas_call(
        kern,
        out_shape=jax.ShapeDtypeStruct((B, C, HW), x.dtype),
        grid_spec=pltpu.PrefetchScalarGridSpec(
            num_scalar_prefetch=1,
            grid=(nb,),
            in_specs=[pl.BlockSpec((B, ct, HW), lambda ci, y_ref: (0, ci, 0)),
                      pl.BlockSpec((NC, C), lambda ci, y_ref: (0, 0)),
                      pl.BlockSpec((NC, C), lambda ci, y_ref: (0, 0))],
            out_specs=pl.BlockSpec((B, ct, HW), lambda ci, y_ref: (0, ci, 0)),
            scratch_shapes=[pltpu.VMEM((nb, B, ct), jnp.float32),
                            pltpu.VMEM((nb, B, ct), jnp.float32)],
        ),
        compiler_params=pltpu.CompilerParams(
            dimension_semantics=("arbitrary",),
            vmem_limit_bytes=_VMEM_LIMIT_BYTES),
    )(y, x3, e0, e1)
    return out3.reshape(B, C, H, W)


def kernel(x, y, embed0, embed1):
    return _cbn(x, y, embed0, embed1, eps=1e-4)


# speedup vs baseline: 1.0242x; 1.0242x over previous
import functools

import jax
import jax.numpy as jnp
from jax import lax
from jax.experimental import pallas as pl
from jax.experimental.pallas import tpu as pltpu

_VMEM_LIMIT_BYTES = 60 << 20


def _cbn_kernel(x_ref, gain_ref, bias_ref, o_ref, *, eps, inv_n):
    B, ct = x_ref.shape[0], x_ref.shape[1]
    ci = pl.program_id(0)
    x = x_ref[...]

    s1 = jnp.sum(x, axis=2, keepdims=True)
    s2 = jnp.sum(x * x, axis=2, keepdims=True)
    mean = jnp.sum(s1, axis=0, keepdims=True) * inv_n
    ex2 = jnp.sum(s2, axis=0, keepdims=True) * inv_n
    var = jnp.maximum(ex2 - mean * mean, 0.0)
    inv_std = lax.rsqrt(var + eps)

    g3 = gain_ref[:, pl.ds(ci, 1), :].reshape(B, ct, 1)
    b3 = bias_ref[:, pl.ds(ci, 1), :].reshape(B, ct, 1)
    scale = g3 * inv_std
    shift = b3 - mean * scale

    o_ref[...] = x * scale + shift


def _pick_channel_tile(B, C, HW, itemsize, target_bytes=8 << 20):
    if C % 8 != 0:
        return C
    per_channel = max(B * HW * itemsize, 1)
    max_ct = (target_bytes // per_channel) // 8 * 8
    max_ct = max(8, min(int(max_ct), C))
    for cand in range(max_ct, 7, -8):
        if C % cand == 0:
            return cand
    return 8


@functools.partial(jax.jit, static_argnames=("eps",))
def _cbn(x, gain, bias, *, eps=1e-4):
    B, C, H, W = x.shape
    HW = H * W
    x3 = x.reshape(B, C, HW)
    ct = _pick_channel_tile(B, C, HW, x.dtype.itemsize)
    nb = C // ct
    g = gain.astype(jnp.float32).reshape(B, nb, ct)
    b = bias.astype(jnp.float32).reshape(B, nb, ct)
    kern = functools.partial(_cbn_kernel, eps=float(eps),
                             inv_n=1.0 / float(B * HW))
    out3 = pl.pallas_call(
        kern,
        out_shape=jax.ShapeDtypeStruct((B, C, HW), x.dtype),
        grid=(nb,),
        in_specs=[pl.BlockSpec((B, ct, HW), lambda ci: (0, ci, 0)),
                  pl.BlockSpec((B, nb, ct), lambda ci: (0, 0, 0)),
                  pl.BlockSpec((B, nb, ct), lambda ci: (0, 0, 0))],
        out_specs=pl.BlockSpec((B, ct, HW), lambda ci: (0, ci, 0)),
        compiler_params=pltpu.CompilerParams(
            dimension_semantics=("parallel",),
            vmem_limit_bytes=_VMEM_LIMIT_BYTES),
    )(x3, g, b)
    return out3.reshape(B, C, H, W)


def kernel(x, y, embed0, embed1):
    gain = 1.0 + jnp.take(embed0, y, axis=0)
    bias = jnp.take(embed1, y, axis=0)
    return _cbn(x, gain, bias, eps=1e-4)
